# Initial kernel scaffold; baseline (speedup 1.0000x reference)
import jax
import jax.numpy as jnp
from jax.experimental import pallas as pl


def kernel(x, edge_index, edge_type, basis, att, root, bias):
    raise NotImplementedError("write your pallas kernel here")


# speedup vs baseline: 3.8687x; 3.8687x over previous
import functools

import jax
import jax.numpy as jnp
from jax import lax
from jax.experimental import pallas as pl
from jax.experimental.pallas import tpu as pltpu
from jax.experimental.pallas import tpu_sc as plsc

_LANES = 16
_N_WORKERS = 32


def _project(x, wcat, bias2, nbd):
    n, d = x.shape
    rows = 2000
    assert n % rows == 0

    def body(x_ref, w_ref, b_ref, y_ref, base_ref):
        ycat = jnp.dot(x_ref[...], w_ref[...], preferred_element_type=jnp.float32)
        y_ref[...] = ycat[:, :nbd]
        base_ref[...] = ycat[:, nbd:] + b_ref[...]

    return pl.pallas_call(
        body,
        grid=(n // rows,),
        in_specs=[
            pl.BlockSpec((rows, d), lambda i: (i, 0)),
            pl.BlockSpec(wcat.shape, lambda i: (0, 0)),
            pl.BlockSpec((1, d), lambda i: (0, 0)),
        ],
        out_specs=[
            pl.BlockSpec((rows, nbd), lambda i: (i, 0)),
            pl.BlockSpec((rows, d), lambda i: (i, 0)),
        ],
        out_shape=(
            jax.ShapeDtypeStruct((n, nbd), jnp.float32),
            jax.ShapeDtypeStruct((n, d), jnp.float32),
        ),
    )(x, wcat, bias2)


def _make_sc_edge_kernel(n, d, nb, nr, e_pad):
    per_w = e_pad // _N_WORKERS
    groups = per_w // _LANES
    nbd = nb * d
    zchunk = n // 10
    assert zchunk % 8 == 0 and per_w % 8 == 0

    mesh = plsc.VectorSubcoreMesh(core_axis_name="c", subcore_axis_name="s")

    @functools.partial(
        pl.kernel,
        mesh=mesh,
        compiler_params=pltpu.CompilerParams(needs_layout_passes=False),
        out_type=(
            jax.ShapeDtypeStruct((2, n, d), jnp.float32),
            jax.ShapeDtypeStruct((2 * n,), jnp.float32),
        ),
        scratch_types=[
            pltpu.VMEM((nr * nb,), jnp.float32),
            pltpu.VMEM((per_w,), jnp.int32),
            pltpu.VMEM((per_w,), jnp.int32),
            pltpu.VMEM((per_w,), jnp.int32),
            pltpu.VMEM((_LANES, nbd), jnp.float32),
            pltpu.VMEM((_LANES, d), jnp.float32),
            pltpu.VMEM((nb * _LANES,), jnp.float32),
            pltpu.VMEM((_LANES,), jnp.float32),
            pltpu.VMEM((1008,), jnp.float32),
            pltpu.VMEM_SHARED((n + 8, d), jnp.float32),
            pltpu.VMEM_SHARED((n + 8,), jnp.float32),
            pltpu.SemaphoreType.DMA,
        ],
    )
    def sc(y_hbm, src_hbm, dst_hbm, et_hbm, att_hbm,
           sums_out, cnt_out,
           att_v, src_v, dst_v, et_v, rows_v, msg_v, ab_v, ones_v,
           zflat_v, sums_sh, cnt_sh, sem):
        cid = lax.axis_index("c")
        sid = lax.axis_index("s")
        w = cid * 16 + sid
        base = pl.multiple_of(w * per_w, 8)

        pltpu.sync_copy(att_hbm, att_v)
        pltpu.sync_copy(src_hbm.at[pl.ds(base, per_w)], src_v)
        pltpu.sync_copy(dst_hbm.at[pl.ds(base, per_w)], dst_v)
        pltpu.sync_copy(et_hbm.at[pl.ds(base, per_w)], et_v)
        ones_v[...] = jnp.ones((_LANES,), jnp.float32)

        @pl.when(sid < 10)
        def _():
            z16 = jnp.zeros((_LANES,), jnp.float32)
            for r in range(_LANES):
                for j in range(d // _LANES):
                    msg_v[r, pl.ds(j * _LANES, _LANES)] = z16

            def zflat_body(k, c):
                zflat_v[pl.ds(k * _LANES, _LANES)] = z16
                return c

            lax.fori_loop(0, 1008 // _LANES, zflat_body, 0)
            off = pl.multiple_of(sid * zchunk, 8)

            def initrows(k, c):
                o = pl.multiple_of(off + k * _LANES, 8)
                pltpu.sync_copy(msg_v, sums_sh.at[pl.ds(o, _LANES)])
                return c

            lax.fori_loop(0, zchunk // _LANES, initrows, 0)
            pltpu.sync_copy(msg_v,
                            sums_sh.at[pl.ds(off + zchunk - _LANES, _LANES)])
            pltpu.sync_copy(zflat_v, cnt_sh.at[pl.ds(off, 1008)])

        plsc.subcore_barrier()

        def grp(g, carry):
            gb = pl.multiple_of(g * _LANES, _LANES)
            s16 = src_v[pl.ds(gb, _LANES)]
            pltpu.async_copy(y_hbm.at[s16], rows_v, sem).wait()
            t16 = et_v[pl.ds(gb, _LANES)]
            for b in range(nb):
                ab_v[pl.ds(b * _LANES, _LANES)] = plsc.load_gather(
                    att_v, [t16 * nb + b])
            for c in range(_LANES):
                a = [
                    plsc.load_gather(
                        ab_v,
                        [jnp.full((_LANES,), b * _LANES + c, jnp.int32)])
                    for b in range(nb)
                ]
                for j in range(d // _LANES):
                    acc = a[0] * rows_v[c, pl.ds(j * _LANES, _LANES)]
                    for b in range(1, nb):
                        acc = acc + a[b] * rows_v[
                            c, pl.ds(b * d + j * _LANES, _LANES)]
                    msg_v[c, pl.ds(j * _LANES, _LANES)] = acc
            d16 = dst_v[pl.ds(gb, _LANES)]
            pltpu.sync_copy(msg_v, sums_sh.at[d16], add=True)
            pltpu.sync_copy(ones_v, cnt_sh.at[d16], add=True)
            return carry

        lax.fori_loop(0, groups, grp, 0)
        plsc.subcore_barrier()

        @pl.when(sid < 10)
        def _():
            off = pl.multiple_of(sid * zchunk, 8)

            def outrows(k, c):
                o = pl.multiple_of(off + k * _LANES, 8)
                pltpu.sync_copy(sums_sh.at[pl.ds(o, _LANES)], msg_v)
                pltpu.sync_copy(msg_v, sums_out.at[cid, pl.ds(o, _LANES)])
                return c

            lax.fori_loop(0, zchunk // _LANES, outrows, 0)
            o2 = pl.multiple_of(off + zchunk - _LANES, 8)
            pltpu.sync_copy(sums_sh.at[pl.ds(o2, _LANES)], msg_v)
            pltpu.sync_copy(msg_v, sums_out.at[cid, pl.ds(o2, _LANES)])
            coff = pl.multiple_of(cid * n + off, 8)
            pltpu.sync_copy(cnt_sh.at[pl.ds(off, zchunk)],
                            zflat_v.at[pl.ds(0, zchunk)])
            pltpu.sync_copy(zflat_v.at[pl.ds(0, zchunk)],
                            cnt_out.at[pl.ds(coff, zchunk)])

    return sc


def _combine(base, sums, cnt3):
    n, d = base.shape
    rows = 2000
    assert n % rows == 0

    def body(base_ref, s_ref, c_ref, o_ref):
        s = s_ref[0] + s_ref[1]
        c = c_ref[0] + c_ref[1]
        o_ref[...] = base_ref[...] + s / jnp.maximum(c, 1.0)

    return pl.pallas_call(
        body,
        grid=(n // rows,),
        in_specs=[
            pl.BlockSpec((rows, d), lambda i: (i, 0)),
            pl.BlockSpec((2, rows, d), lambda i: (0, i, 0)),
            pl.BlockSpec((2, rows, 1), lambda i: (0, i, 0)),
        ],
        out_specs=pl.BlockSpec((rows, d), lambda i: (i, 0)),
        out_shape=jax.ShapeDtypeStruct((n, d), jnp.float32),
    )(base, sums, cnt3)


def kernel(x, edge_index, edge_type, basis, att, root, bias):
    n, d = x.shape
    nb = basis.shape[0]
    e = edge_type.shape[0]
    nbd = nb * d

    src = edge_index[0].astype(jnp.int32)
    dst = edge_index[1].astype(jnp.int32)
    et = edge_type.astype(jnp.int32)

    w2 = basis.transpose(1, 0, 2).reshape(d, nbd)
    wcat = jnp.concatenate([w2, root], axis=1)
    bias2 = bias.reshape(1, d)

    y, base = _project(x, wcat, bias2, nbd)

    e_pad = -(-e // (_N_WORKERS * _LANES)) * (_N_WORKERS * _LANES)
    pad = e_pad - e
    src_p = jnp.concatenate([src, jnp.zeros((pad,), jnp.int32)])
    dst_p = jnp.concatenate([dst, jnp.full((pad,), n, jnp.int32)])
    et_p = jnp.concatenate([et, jnp.zeros((pad,), jnp.int32)])
    att_flat = att.reshape(-1)

    sc_fn = _make_sc_edge_kernel(n, d, nb, att.shape[0], e_pad)
    sums, cnt = sc_fn(y, src_p, dst_p, et_p, att_flat)

    return _combine(base, sums, cnt.reshape(2, n, 1))


# speedup vs baseline: 6.2824x; 1.6239x over previous
import functools

import jax
import jax.numpy as jnp
from jax import lax
from jax.experimental import pallas as pl
from jax.experimental.pallas import tpu as pltpu
from jax.experimental.pallas import tpu_sc as plsc

_LANES = 16
_N_WORKERS = 32


def _project(x, wcat, bias2, nbd):
    n, d = x.shape
    rows = 2000
    assert n % rows == 0

    def body(x_ref, w_ref, b_ref, y_ref, base_ref):
        ycat = jnp.dot(x_ref[...], w_ref[...], preferred_element_type=jnp.float32)
        y_ref[...] = ycat[:, :nbd]
        base_ref[...] = ycat[:, nbd:] + b_ref[...]

    return pl.pallas_call(
        body,
        grid=(n // rows,),
        in_specs=[
            pl.BlockSpec((rows, d), lambda i: (i, 0)),
            pl.BlockSpec(wcat.shape, lambda i: (0, 0)),
            pl.BlockSpec((1, d), lambda i: (0, 0)),
        ],
        out_specs=[
            pl.BlockSpec((rows, nbd), lambda i: (i, 0)),
            pl.BlockSpec((rows, d), lambda i: (i, 0)),
        ],
        out_shape=(
            jax.ShapeDtypeStruct((n, nbd), jnp.float32),
            jax.ShapeDtypeStruct((n, d), jnp.float32),
        ),
    )(x, wcat, bias2)


def _make_sc_edge_kernel(n, d, nb, nr, e_pad):
    per_w = e_pad // _N_WORKERS
    groups = per_w // _LANES
    nbd = nb * d
    zchunk = n // 10
    assert zchunk % 8 == 0 and per_w % 8 == 0

    mesh = plsc.VectorSubcoreMesh(core_axis_name="c", subcore_axis_name="s")

    @functools.partial(
        pl.kernel,
        mesh=mesh,
        compiler_params=pltpu.CompilerParams(needs_layout_passes=False),
        out_type=(
            jax.ShapeDtypeStruct((2, n, d), jnp.float32),
            jax.ShapeDtypeStruct((2 * n,), jnp.float32),
        ),
        scratch_types=[
            pltpu.VMEM((nr * nb,), jnp.float32),
            pltpu.VMEM((per_w,), jnp.int32),
            pltpu.VMEM((per_w,), jnp.int32),
            pltpu.VMEM((per_w,), jnp.int32),
            pltpu.VMEM((_LANES, nbd), jnp.float32),
            pltpu.VMEM((_LANES, nbd), jnp.float32),
            pltpu.VMEM((_LANES, d), jnp.float32),
            pltpu.VMEM((nb * _LANES,), jnp.float32),
            pltpu.VMEM((_LANES,), jnp.float32),
            pltpu.VMEM((1008,), jnp.float32),
            pltpu.VMEM_SHARED((n + 8, d), jnp.float32),
            pltpu.VMEM_SHARED((n + 8,), jnp.float32),
            pltpu.SemaphoreType.DMA,
            pltpu.SemaphoreType.DMA,
        ],
    )
    def sc(y_hbm, src_hbm, dst_hbm, et_hbm, att_hbm,
           sums_out, cnt_out,
           att_v, src_v, dst_v, et_v, rows_a, rows_b, msg_v, ab_v, ones_v,
           zflat_v, sums_sh, cnt_sh, sem_a, sem_b):
        cid = lax.axis_index("c")
        sid = lax.axis_index("s")
        w = cid * 16 + sid
        base = pl.multiple_of(w * per_w, 8)

        pltpu.sync_copy(att_hbm, att_v)
        pltpu.sync_copy(src_hbm.at[pl.ds(base, per_w)], src_v)
        pltpu.sync_copy(dst_hbm.at[pl.ds(base, per_w)], dst_v)
        pltpu.sync_copy(et_hbm.at[pl.ds(base, per_w)], et_v)
        ones_v[...] = jnp.ones((_LANES,), jnp.float32)

        @pl.when(sid < 10)
        def _():
            z16 = jnp.zeros((_LANES,), jnp.float32)
            for r in range(_LANES):
                for j in range(d // _LANES):
                    msg_v[r, pl.ds(j * _LANES, _LANES)] = z16

            def zflat_body(k, c):
                zflat_v[pl.ds(k * _LANES, _LANES)] = z16
                return c

            lax.fori_loop(0, 1008 // _LANES, zflat_body, 0)
            off = pl.multiple_of(sid * zchunk, 8)

            def initrows(k, c):
                o = pl.multiple_of(off + k * _LANES, 8)
                pltpu.sync_copy(msg_v, sums_sh.at[pl.ds(o, _LANES)])
                return c

            lax.fori_loop(0, zchunk // _LANES, initrows, 0)
            pltpu.sync_copy(msg_v,
                            sums_sh.at[pl.ds(off + zchunk - _LANES, _LANES)])
            pltpu.sync_copy(zflat_v, cnt_sh.at[pl.ds(off, 1008)])

        plsc.subcore_barrier()

        rows = (rows_a, rows_b)
        sems = (sem_a, sem_b)

        def issue(gg, buf, s):
            gb = pl.multiple_of(gg * _LANES, _LANES)
            pltpu.async_copy(y_hbm.at[src_v[pl.ds(gb, _LANES)]], buf, s)

        def compute(gg, buf):
            gb = pl.multiple_of(gg * _LANES, _LANES)
            t16 = et_v[pl.ds(gb, _LANES)]
            for b in range(nb):
                ab_v[pl.ds(b * _LANES, _LANES)] = plsc.load_gather(
                    att_v, [t16 * nb + b])
            for c in range(_LANES):
                a = [
                    plsc.load_gather(
                        ab_v,
                        [jnp.full((_LANES,), b * _LANES + c, jnp.int32)])
                    for b in range(nb)
                ]
                for j in range(d // _LANES):
                    acc = a[0] * buf[c, pl.ds(j * _LANES, _LANES)]
                    for b in range(1, nb):
                        acc = acc + a[b] * buf[
                            c, pl.ds(b * d + j * _LANES, _LANES)]
                    msg_v[c, pl.ds(j * _LANES, _LANES)] = acc
            d16 = dst_v[pl.ds(gb, _LANES)]
            pltpu.sync_copy(msg_v, sums_sh.at[d16], add=True)
            pltpu.sync_copy(ones_v, cnt_sh.at[d16], add=True)

        issue(0, rows[0], sems[0])

        def pair(p, carry):
            g = p * 2
            for b in range(2):
                gg = g + b

                @pl.when(gg + 1 < groups)
                def _():
                    issue(gg + 1, rows[1 - b], sems[1 - b])

                pltpu.make_async_copy(
                    y_hbm.at[src_v[pl.ds(0, _LANES)]], rows[b],
                    sems[b]).wait()
                compute(gg, rows[b])
            return carry

        lax.fori_loop(0, groups // 2, pair, 0)
        plsc.subcore_barrier()

        @pl.when(sid < 10)
        def _():
            off = pl.multiple_of(sid * zchunk, 8)

            def outrows(k, c):
                o = pl.multiple_of(off + k * _LANES, 8)
                pltpu.sync_copy(sums_sh.at[pl.ds(o, _LANES)], msg_v)
                pltpu.sync_copy(msg_v, sums_out.at[cid, pl.ds(o, _LANES)])
                return c

            lax.fori_loop(0, zchunk // _LANES, outrows, 0)
            o2 = pl.multiple_of(off + zchunk - _LANES, 8)
            pltpu.sync_copy(sums_sh.at[pl.ds(o2, _LANES)], msg_v)
            pltpu.sync_copy(msg_v, sums_out.at[cid, pl.ds(o2, _LANES)])
            coff = pl.multiple_of(cid * n + off, 8)
            pltpu.sync_copy(cnt_sh.at[pl.ds(off, zchunk)],
                            zflat_v.at[pl.ds(0, zchunk)])
            pltpu.sync_copy(zflat_v.at[pl.ds(0, zchunk)],
                            cnt_out.at[pl.ds(coff, zchunk)])

    return sc


def _combine(base, sums, cnt3):
    n, d = base.shape
    rows = 2000
    assert n % rows == 0

    def body(base_ref, s_ref, c_ref, o_ref):
        s = s_ref[0] + s_ref[1]
        c = c_ref[0] + c_ref[1]
        o_ref[...] = base_ref[...] + s / jnp.maximum(c, 1.0)

    return pl.pallas_call(
        body,
        grid=(n // rows,),
        in_specs=[
            pl.BlockSpec((rows, d), lambda i: (i, 0)),
            pl.BlockSpec((2, rows, d), lambda i: (0, i, 0)),
            pl.BlockSpec((2, rows, 1), lambda i: (0, i, 0)),
        ],
        out_specs=pl.BlockSpec((rows, d), lambda i: (i, 0)),
        out_shape=jax.ShapeDtypeStruct((n, d), jnp.float32),
    )(base, sums, cnt3)


def kernel(x, edge_index, edge_type, basis, att, root, bias):
    n, d = x.shape
    nb = basis.shape[0]
    e = edge_type.shape[0]
    nbd = nb * d

    src = edge_index[0].astype(jnp.int32)
    dst = edge_index[1].astype(jnp.int32)
    et = edge_type.astype(jnp.int32)

    w2 = basis.transpose(1, 0, 2).reshape(d, nbd)
    wcat = jnp.concatenate([w2, root], axis=1)
    bias2 = bias.reshape(1, d)

    y, base = _project(x, wcat, bias2, nbd)

    e_pad = -(-e // (_N_WORKERS * _LANES * 2)) * (_N_WORKERS * _LANES * 2)
    pad = e_pad - e
    src_p = jnp.concatenate([src, jnp.zeros((pad,), jnp.int32)])
    dst_p = jnp.concatenate([dst, jnp.full((pad,), n, jnp.int32)])
    et_p = jnp.concatenate([et, jnp.zeros((pad,), jnp.int32)])
    att_flat = att.reshape(-1)

    sc_fn = _make_sc_edge_kernel(n, d, nb, att.shape[0], e_pad)
    sums, cnt = sc_fn(y, src_p, dst_p, et_p, att_flat)

    return _combine(base, sums, cnt.reshape(2, n, 1))


# speedup vs baseline: 6.6725x; 1.0621x over previous
import functools

import jax
import jax.numpy as jnp
from jax import lax
from jax.experimental import pallas as pl
from jax.experimental.pallas import tpu as pltpu
from jax.experimental.pallas import tpu_sc as plsc

_LANES = 16
_N_WORKERS = 32


def _project(x, wcat, bias2, nbd):
    n, d = x.shape
    rows = 2000
    assert n % rows == 0

    def body(x_ref, w_ref, b_ref, y_ref, base_ref):
        ycat = jnp.dot(x_ref[...], w_ref[...], preferred_element_type=jnp.float32)
        y_ref[...] = ycat[:, :nbd]
        base_ref[...] = ycat[:, nbd:] + b_ref[...]

    return pl.pallas_call(
        body,
        grid=(n // rows,),
        in_specs=[
            pl.BlockSpec((rows, d), lambda i: (i, 0)),
            pl.BlockSpec(wcat.shape, lambda i: (0, 0)),
            pl.BlockSpec((1, d), lambda i: (0, 0)),
        ],
        out_specs=[
            pl.BlockSpec((rows, nbd), lambda i: (i, 0)),
            pl.BlockSpec((rows, d), lambda i: (i, 0)),
        ],
        out_shape=(
            jax.ShapeDtypeStruct((n, nbd), jnp.float32),
            jax.ShapeDtypeStruct((n, d), jnp.float32),
        ),
    )(x, wcat, bias2)


def _make_sc_edge_kernel(n, d, nb, nr, e_pad):
    per_w = e_pad // _N_WORKERS
    groups = per_w // _LANES
    nbd = nb * d
    zchunk = n // 10
    assert zchunk % 8 == 0 and per_w % 8 == 0

    mesh = plsc.VectorSubcoreMesh(core_axis_name="c", subcore_axis_name="s")

    @functools.partial(
        pl.kernel,
        mesh=mesh,
        compiler_params=pltpu.CompilerParams(needs_layout_passes=False),
        out_type=(
            jax.ShapeDtypeStruct((2, n, d), jnp.float32),
            jax.ShapeDtypeStruct((2 * n,), jnp.float32),
        ),
        scratch_types=[
            pltpu.VMEM((nr * nb,), jnp.float32),
            pltpu.VMEM((per_w,), jnp.int32),
            pltpu.VMEM((per_w,), jnp.int32),
            pltpu.VMEM((per_w,), jnp.int32),
            pltpu.VMEM((_LANES, nbd), jnp.float32),
            pltpu.VMEM((_LANES, nbd), jnp.float32),
            pltpu.VMEM((_LANES, d), jnp.float32),
            pltpu.VMEM((_LANES, d), jnp.float32),
            pltpu.VMEM((nb * _LANES,), jnp.float32),
            pltpu.VMEM((per_w,), jnp.float32),
            pltpu.VMEM((1008,), jnp.float32),
            pltpu.VMEM_SHARED((n + 8, d), jnp.float32),
            pltpu.VMEM_SHARED((n + 8,), jnp.float32),
            pltpu.SemaphoreType.DMA,
            pltpu.SemaphoreType.DMA,
            pltpu.SemaphoreType.DMA,
            pltpu.SemaphoreType.DMA,
        ],
    )
    def sc(y_hbm, src_hbm, dst_hbm, et_hbm, att_hbm,
           sums_out, cnt_out,
           att_v, src_v, dst_v, et_v, rows_a, rows_b, msg_a, msg_b, ab_v,
           ones_v, zflat_v, sums_sh, cnt_sh, sem_a, sem_b, sem_ma, sem_mb):
        cid = lax.axis_index("c")
        sid = lax.axis_index("s")
        w = cid * 16 + sid
        base = pl.multiple_of(w * per_w, 8)

        pltpu.sync_copy(att_hbm, att_v)
        pltpu.sync_copy(src_hbm.at[pl.ds(base, per_w)], src_v)
        pltpu.sync_copy(dst_hbm.at[pl.ds(base, per_w)], dst_v)
        pltpu.sync_copy(et_hbm.at[pl.ds(base, per_w)], et_v)

        def fill_ones(k, c):
            ones_v[pl.ds(pl.multiple_of(k * _LANES, _LANES), _LANES)] = (
                jnp.ones((_LANES,), jnp.float32))
            return c

        lax.fori_loop(0, per_w // _LANES, fill_ones, 0)

        @pl.when(sid < 10)
        def _():
            z16 = jnp.zeros((_LANES,), jnp.float32)
            for r in range(_LANES):
                for j in range(d // _LANES):
                    msg_a[r, pl.ds(j * _LANES, _LANES)] = z16

            def zflat_body(k, c):
                zflat_v[pl.ds(k * _LANES, _LANES)] = z16
                return c

            lax.fori_loop(0, 1008 // _LANES, zflat_body, 0)
            off = pl.multiple_of(sid * zchunk, 8)

            def initrows(k, c):
                o = pl.multiple_of(off + k * _LANES, 8)
                pltpu.sync_copy(msg_a, sums_sh.at[pl.ds(o, _LANES)])
                return c

            lax.fori_loop(0, zchunk // _LANES, initrows, 0)
            pltpu.sync_copy(msg_a,
                            sums_sh.at[pl.ds(off + zchunk - _LANES, _LANES)])
            pltpu.sync_copy(zflat_v, cnt_sh.at[pl.ds(off, 1008)])

        plsc.subcore_barrier()

        rows = (rows_a, rows_b)
        sems = (sem_a, sem_b)
        msgs = (msg_a, msg_b)
        msems = (sem_ma, sem_mb)

        def issue(gg, buf, s):
            gb = pl.multiple_of(gg * _LANES, _LANES)
            pltpu.async_copy(y_hbm.at[src_v[pl.ds(gb, _LANES)]], buf, s)

        def compute(gg, buf, msg):
            gb = pl.multiple_of(gg * _LANES, _LANES)
            t16 = et_v[pl.ds(gb, _LANES)]
            for b in range(nb):
                ab_v[pl.ds(b * _LANES, _LANES)] = plsc.load_gather(
                    att_v, [t16 * nb + b])
            for c in range(_LANES):
                a = [
                    plsc.load_gather(
                        ab_v,
                        [jnp.full((_LANES,), b * _LANES + c, jnp.int32)])
                    for b in range(nb)
                ]
                for j in range(d // _LANES):
                    acc = a[0] * buf[c, pl.ds(j * _LANES, _LANES)]
                    for b in range(1, nb):
                        acc = acc + a[b] * buf[
                            c, pl.ds(b * d + j * _LANES, _LANES)]
                    msg[c, pl.ds(j * _LANES, _LANES)] = acc

        def wait_gather(b):
            pltpu.make_async_copy(
                y_hbm.at[src_v[pl.ds(0, _LANES)]], rows[b], sems[b]).wait()

        def wait_scatter(b):
            pltpu.make_async_copy(
                msgs[b], sums_sh.at[dst_v[pl.ds(0, _LANES)]],
                msems[b]).wait()

        issue(0, rows[0], sems[0])

        def pair(p, carry):
            g = p * 2
            for b in range(2):
                gg = g + b

                @pl.when(gg + 1 < groups)
                def _():
                    issue(gg + 1, rows[1 - b], sems[1 - b])

                wait_gather(b)

                @pl.when(gg >= 2)
                def _():
                    wait_scatter(b)

                compute(gg, rows[b], msgs[b])
                gb = pl.multiple_of(gg * _LANES, _LANES)
                d16 = dst_v[pl.ds(gb, _LANES)]
                pltpu.async_copy(msgs[b], sums_sh.at[d16], msems[b],
                                 add=True)
            return carry

        lax.fori_loop(0, groups // 2, pair, 0)
        wait_scatter(0)
        wait_scatter(1)
        pltpu.sync_copy(ones_v, cnt_sh.at[dst_v], add=True)
        plsc.subcore_barrier()

        @pl.when(sid < 10)
        def _():
            off = pl.multiple_of(sid * zchunk, 8)

            def outrows(k, c):
                o = pl.multiple_of(off + k * _LANES, 8)
                pltpu.sync_copy(sums_sh.at[pl.ds(o, _LANES)], msg_a)
                pltpu.sync_copy(msg_a, sums_out.at[cid, pl.ds(o, _LANES)])
                return c

            lax.fori_loop(0, zchunk // _LANES, outrows, 0)
            o2 = pl.multiple_of(off + zchunk - _LANES, 8)
            pltpu.sync_copy(sums_sh.at[pl.ds(o2, _LANES)], msg_a)
            pltpu.sync_copy(msg_a, sums_out.at[cid, pl.ds(o2, _LANES)])
            coff = pl.multiple_of(cid * n + off, 8)
            pltpu.sync_copy(cnt_sh.at[pl.ds(off, zchunk)],
                            zflat_v.at[pl.ds(0, zchunk)])
            pltpu.sync_copy(zflat_v.at[pl.ds(0, zchunk)],
                            cnt_out.at[pl.ds(coff, zchunk)])

    return sc


def _combine(base, sums, cnt3):
    n, d = base.shape
    rows = 2000
    assert n % rows == 0

    def body(base_ref, s_ref, c_ref, o_ref):
        s = s_ref[0] + s_ref[1]
        c = c_ref[0] + c_ref[1]
        o_ref[...] = base_ref[...] + s / jnp.maximum(c, 1.0)

    return pl.pallas_call(
        body,
        grid=(n // rows,),
        in_specs=[
            pl.BlockSpec((rows, d), lambda i: (i, 0)),
            pl.BlockSpec((2, rows, d), lambda i: (0, i, 0)),
            pl.BlockSpec((2, rows, 1), lambda i: (0, i, 0)),
        ],
        out_specs=pl.BlockSpec((rows, d), lambda i: (i, 0)),
        out_shape=jax.ShapeDtypeStruct((n, d), jnp.float32),
    )(base, sums, cnt3)


def kernel(x, edge_index, edge_type, basis, att, root, bias):
    n, d = x.shape
    nb = basis.shape[0]
    e = edge_type.shape[0]
    nbd = nb * d

    src = edge_index[0].astype(jnp.int32)
    dst = edge_index[1].astype(jnp.int32)
    et = edge_type.astype(jnp.int32)

    w2 = basis.transpose(1, 0, 2).reshape(d, nbd)
    wcat = jnp.concatenate([w2, root], axis=1)
    bias2 = bias.reshape(1, d)

    y, base = _project(x, wcat, bias2, nbd)

    e_pad = -(-e // (_N_WORKERS * _LANES * 2)) * (_N_WORKERS * _LANES * 2)
    pad = e_pad - e
    src_p = jnp.concatenate([src, jnp.zeros((pad,), jnp.int32)])
    dst_p = jnp.concatenate([dst, jnp.full((pad,), n, jnp.int32)])
    et_p = jnp.concatenate([et, jnp.zeros((pad,), jnp.int32)])
    att_flat = att.reshape(-1)

    sc_fn = _make_sc_edge_kernel(n, d, nb, att.shape[0], e_pad)
    sums, cnt = sc_fn(y, src_p, dst_p, et_p, att_flat)

    return _combine(base, sums, cnt.reshape(2, n, 1))
